# baseline (speedup 1.0000x reference)
import functools

import jax
import jax.numpy as jnp
from jax import lax
from jax.experimental import pallas as pl
from jax.experimental.pallas import tpu as pltpu
from jax.experimental.pallas import tpu_sc as plsc

HIDDEN_DIM = 512
N_EXPERTS = 16
TOP_K = 2
SEQ = 2048
IN_DIM = 1024

BLK = 256
ROWS = SEQ * TOP_K
P = ROWS
NBLOCKS = ROWS // BLK
NI = NBLOCKS + N_EXPERTS

_NW = 32
_TPW = SEQ // _NW


def _sc_scatter_tokens(inp, dt):
    mesh = plsc.VectorSubcoreMesh(core_axis_name="c", subcore_axis_name="s")
    info = plsc.get_sparse_core_info()

    @functools.partial(
        pl.kernel,
        mesh=mesh,
        out_type=jax.ShapeDtypeStruct((P, IN_DIM), jnp.float32),
        scratch_types=[
            pltpu.VMEM((_TPW,), jnp.int32),
            pltpu.VMEM((_TPW,), jnp.int32),
            pltpu.VMEM((_TPW, IN_DIM), jnp.float32),
            pltpu.SemaphoreType.DMA,
            pltpu.SemaphoreType.DMA,
        ],
    )
    def k(inp_hbm, dt_hbm, out_hbm, idx0_v, idx1_v, rows_v, s0, s1):
        wid = lax.axis_index("s") * info.num_cores + lax.axis_index("c")
        r = wid // 2
        c0 = (wid % 2) * _TPW
        pltpu.sync_copy(dt_hbm.at[r, pl.ds(c0, _TPW)], idx0_v)
        pltpu.sync_copy(dt_hbm.at[_SR + r, pl.ds(c0, _TPW)], idx1_v)
        pltpu.sync_copy(inp_hbm.at[pl.ds(wid * _TPW, _TPW)], rows_v)
        cp0 = pltpu.async_copy(rows_v, out_hbm.at[idx0_v], s0)
        cp1 = pltpu.async_copy(rows_v, out_hbm.at[idx1_v], s1)
        cp0.wait()
        cp1.wait()

    return k(inp, dt)


def _sc_gather_rows(src, dt, n_rows, d):
    info = plsc.get_sparse_core_info()
    per_w = 128
    chunk = 64
    n_chunks = per_w // chunk
    mesh = plsc.VectorSubcoreMesh(core_axis_name="c", subcore_axis_name="s")

    @functools.partial(
        pl.kernel,
        mesh=mesh,
        out_type=jax.ShapeDtypeStruct((n_rows, d), jnp.float32),
        scratch_types=[
            pltpu.VMEM((per_w,), jnp.int32),
            pltpu.VMEM((chunk, d), jnp.float32),
            pltpu.SemaphoreType.DMA,
        ],
    )
    def k(src_hbm, dt_hbm, out_hbm, idx_v, rows_v, sem):
        wid = lax.axis_index("s") * info.num_cores + lax.axis_index("c")
        base = wid * per_w
        pltpu.sync_copy(dt_hbm.at[wid], idx_v)
        for c in range(n_chunks):
            pltpu.async_copy(src_hbm.at[idx_v.at[pl.ds(c * chunk, chunk)]],
                             rows_v, sem).wait()
            pltpu.sync_copy(rows_v, out_hbm.at[pl.ds(base + c * chunk, chunk)])

    return k(src, dt)


def _ffn_body(meta_ref, x_ref, wu_ref, wg_ref, wd_ref, o_ref):
    i = pl.program_id(0)
    lo = meta_ref[2, i]
    hi = meta_ref[3, i]

    @pl.when(hi > lo)
    def _():
        x = x_ref[...].astype(jnp.bfloat16)
        wu = wu_ref[0].astype(jnp.bfloat16)
        wg = wg_ref[0].astype(jnp.bfloat16)
        wd = wd_ref[0].astype(jnp.bfloat16)
        h = lax.dot_general(x, wu, (((1,), (1,)), ((), ())),
                            preferred_element_type=jnp.float32)
        g = lax.dot_general(x, wg, (((1,), (1,)), ((), ())),
                            preferred_element_type=jnp.float32)
        a = (jax.nn.gelu(g) * h).astype(jnp.bfloat16)
        y = lax.dot_general(a, wd, (((1,), (0,)), ((), ())),
                            preferred_element_type=jnp.float32)
        ri = lax.broadcasted_iota(jnp.int32, (BLK, IN_DIM), 0)
        o_ref[...] = jnp.where((ri >= lo) & (ri < hi), y, o_ref[...])


def _grouped_ffn(x_perm, meta, wu, wg, wd):
    xmap = lambda i, mt: (mt[0, i], 0)
    wmap = lambda i, mt: (mt[1, i], 0, 0)
    grid_spec = pltpu.PrefetchScalarGridSpec(
        num_scalar_prefetch=1,
        grid=(NI,),
        in_specs=[
            pl.BlockSpec((BLK, IN_DIM), xmap),
            pl.BlockSpec((1, HIDDEN_DIM, IN_DIM), wmap),
            pl.BlockSpec((1, HIDDEN_DIM, IN_DIM), wmap),
            pl.BlockSpec((1, HIDDEN_DIM, IN_DIM), wmap),
        ],
        out_specs=pl.BlockSpec((BLK, IN_DIM), xmap),
    )
    return pl.pallas_call(
        _ffn_body,
        grid_spec=grid_spec,
        out_shape=jax.ShapeDtypeStruct((P, IN_DIM), jnp.float32),
    )(meta, x_perm, wu, wg, wd)


def _combine_body(y0_ref, y1_ref, w_ref, o_ref):
    o_ref[...] = (y0_ref[...] * w_ref[:, 0:1] + y1_ref[...] * w_ref[:, 1:2])


def _combine(y_unperm, weights):
    rows = 256
    nblk = SEQ // rows
    return pl.pallas_call(
        _combine_body,
        grid=(nblk,),
        in_specs=[
            pl.BlockSpec((rows, IN_DIM), lambda i: (i, 0)),
            pl.BlockSpec((rows, IN_DIM), lambda i: (i + nblk, 0)),
            pl.BlockSpec((rows, TOP_K), lambda i: (i, 0)),
        ],
        out_specs=pl.BlockSpec((rows, IN_DIM), lambda i: (i, 0)),
        out_shape=jax.ShapeDtypeStruct((SEQ, IN_DIM), jnp.float32),
    )(y_unperm, y_unperm, weights)


_SR = SEQ // 128


def _routing_body(s0_ref, s1_ref, dt_ref, meta_ref):
    s0 = s0_ref[...]
    s1 = s1_ref[...]
    lane = lax.broadcasted_iota(jnp.int32, (_SR, 128), 1)
    row = lax.broadcasted_iota(jnp.int32, (_SR, 128), 0)
    ucol = lax.broadcasted_iota(jnp.int32, (128, 128), 1)
    urow = lax.broadcasted_iota(jnp.int32, (128, 128), 0)
    upper = (urow < ucol).astype(jnp.bfloat16)
    lrow = lax.broadcasted_iota(jnp.int32, (_SR, _SR), 0)
    lcol = lax.broadcasted_iota(jnp.int32, (_SR, _SR), 1)
    lower = (lrow > lcol).astype(jnp.bfloat16)
    ones = jnp.ones((128, 128), jnp.bfloat16)

    def eprefix(m):
        mb = m.astype(jnp.bfloat16)
        pl_lane = lax.dot_general(mb, upper, (((1,), (0,)), ((), ())),
                                  preferred_element_type=jnp.float32)
        tot_rep = lax.dot_general(mb, ones, (((1,), (0,)), ((), ())),
                                  preferred_element_type=jnp.float32)
        carry = lax.dot_general(lower, tot_rep.astype(jnp.bfloat16),
                                (((1,), (0,)), ((), ())),
                                preferred_element_type=jnp.float32)
        return pl_lane + carry

    d0 = jnp.zeros((_SR, 128), jnp.float32)
    d1 = jnp.zeros((_SR, 128), jnp.float32)
    end_prev = jnp.int32(0)
    starts, ends = [], []
    for e in range(N_EXPERTS):
        m0 = (s0 == e)
        m1 = (s1 == e)
        p0 = eprefix(m0)
        p1 = eprefix(m1)
        cnt = (jnp.sum(m0.astype(jnp.float32))
               + jnp.sum(m1.astype(jnp.float32))).astype(jnp.int32)
        starts.append(end_prev)
        end_prev = end_prev + cnt
        ends.append(end_prev)
        base = starts[e].astype(jnp.float32) + p0 + p1
        d0 = jnp.where(m0, base, d0)
        d1 = jnp.where(m1, base + m0.astype(jnp.float32), d1)
    dt_ref[0:_SR, :] = d0.astype(jnp.int32)
    dt_ref[_SR:2 * _SR, :] = d1.astype(jnp.int32)
    lane1 = lax.broadcasted_iota(jnp.int32, (1, 128), 1)
    slot = jnp.int32(0)
    last_e = jnp.int32(0)
    sel_cond = []
    vals = []
    for b in range(NBLOCKS):
        for e in range(N_EXPERTS):
            cs, ce = starts[e], ends[e]
            present = ((cs < (b + 1) * BLK) & (ce > b * BLK) & (ce > cs))
            lo = jnp.maximum(cs - b * BLK, 0)
            hi = jnp.minimum(ce - b * BLK, BLK)
            sel_cond.append((lane1 == slot) & present)
            vals.append((jnp.int32(b), jnp.int32(e), lo, hi))
            slot = slot + present.astype(jnp.int32)
            last_e = jnp.where(present, e, last_e)
    fields = [jnp.full((1, 128), NBLOCKS - 1, jnp.int32),
              jnp.full((1, 128), 0, jnp.int32) + last_e,
              jnp.zeros((1, 128), jnp.int32),
              jnp.zeros((1, 128), jnp.int32)]
    for cond, v in zip(sel_cond, vals):
        for f in range(4):
            fields[f] = jnp.where(cond, v[f], fields[f])
    meta_ref[0:1, :] = fields[0]
    meta_ref[1:2, :] = fields[1]
    meta_ref[2:3, :] = fields[2]
    meta_ref[3:4, :] = fields[3]


def _routing(selections):
    s0 = selections[:, 0].reshape(_SR, 128)
    s1 = selections[:, 1].reshape(_SR, 128)
    dt, meta = pl.pallas_call(
        _routing_body,
        out_shape=(jax.ShapeDtypeStruct((2 * _SR, 128), jnp.int32),
                   jax.ShapeDtypeStruct((4, 128), jnp.int32)),
    )(s0, s1)
    return dt, meta


def kernel(inp, weights, selections, up_proj, gate_proj, down_proj):
    dt, meta = _routing(selections)
    wu = up_proj.reshape(N_EXPERTS, HIDDEN_DIM, IN_DIM)
    wg = gate_proj.reshape(N_EXPERTS, HIDDEN_DIM, IN_DIM)
    wd = down_proj.reshape(N_EXPERTS, HIDDEN_DIM, IN_DIM)
    x_perm = _sc_scatter_tokens(inp, dt)
    y = _grouped_ffn(x_perm, meta, wu, wg, wd)
    y_unperm = _sc_gather_rows(y, dt, ROWS, IN_DIM)
    return _combine(y_unperm, weights)


# speedup vs baseline: 1.1580x; 1.1580x over previous
import functools

import jax
import jax.numpy as jnp
from jax import lax
from jax.experimental import pallas as pl
from jax.experimental.pallas import tpu as pltpu
from jax.experimental.pallas import tpu_sc as plsc

HIDDEN_DIM = 512
N_EXPERTS = 16
TOP_K = 2
SEQ = 2048
IN_DIM = 1024

BLK = 512
ROWS = SEQ * TOP_K
NB = ROWS // BLK + N_EXPERTS
P = NB * BLK

_NW = 32
_TPW = SEQ // _NW


def _sc_scatter_tokens(inp, dt):
    mesh = plsc.VectorSubcoreMesh(core_axis_name="c", subcore_axis_name="s")
    info = plsc.get_sparse_core_info()

    @functools.partial(
        pl.kernel,
        mesh=mesh,
        out_type=jax.ShapeDtypeStruct((P, IN_DIM), jnp.float32),
        scratch_types=[
            pltpu.VMEM((_TPW,), jnp.int32),
            pltpu.VMEM((_TPW,), jnp.int32),
            pltpu.VMEM((_TPW, IN_DIM), jnp.float32),
            pltpu.SemaphoreType.DMA,
            pltpu.SemaphoreType.DMA,
        ],
    )
    def k(inp_hbm, dt_hbm, out_hbm, idx0_v, idx1_v, rows_v, s0, s1):
        wid = lax.axis_index("s") * info.num_cores + lax.axis_index("c")
        r = wid // 2
        c0 = (wid % 2) * _TPW
        pltpu.sync_copy(dt_hbm.at[r, pl.ds(c0, _TPW)], idx0_v)
        pltpu.sync_copy(dt_hbm.at[_SR + r, pl.ds(c0, _TPW)], idx1_v)
        pltpu.sync_copy(inp_hbm.at[pl.ds(wid * _TPW, _TPW)], rows_v)
        cp0 = pltpu.async_copy(rows_v, out_hbm.at[idx0_v], s0)
        cp1 = pltpu.async_copy(rows_v, out_hbm.at[idx1_v], s1)
        cp0.wait()
        cp1.wait()

    return k(inp, dt)


def _sc_gather_rows(src, dt, n_rows, d):
    info = plsc.get_sparse_core_info()
    per_w = 128
    chunk = 64
    n_chunks = per_w // chunk
    mesh = plsc.VectorSubcoreMesh(core_axis_name="c", subcore_axis_name="s")

    @functools.partial(
        pl.kernel,
        mesh=mesh,
        out_type=jax.ShapeDtypeStruct((n_rows, d), jnp.float32),
        scratch_types=[
            pltpu.VMEM((per_w,), jnp.int32),
            pltpu.VMEM((chunk, d), jnp.float32),
            pltpu.SemaphoreType.DMA,
        ],
    )
    def k(src_hbm, dt_hbm, out_hbm, idx_v, rows_v, sem):
        wid = lax.axis_index("s") * info.num_cores + lax.axis_index("c")
        base = wid * per_w
        pltpu.sync_copy(dt_hbm.at[wid], idx_v)
        for c in range(n_chunks):
            pltpu.async_copy(src_hbm.at[idx_v.at[pl.ds(c * chunk, chunk)]],
                             rows_v, sem).wait()
            pltpu.sync_copy(rows_v, out_hbm.at[pl.ds(base + c * chunk, chunk)])

    return k(src, dt)


def _ffn_body(meta_ref, x_ref, wu_ref, wg_ref, wd_ref, o_ref):
    b = pl.program_id(0)

    @pl.when(b < meta_ref[NB])
    def _():
        x = x_ref[...].astype(jnp.bfloat16)
        wu = wu_ref[0].astype(jnp.bfloat16)
        wg = wg_ref[0].astype(jnp.bfloat16)
        wd = wd_ref[0].astype(jnp.bfloat16)
        h = lax.dot_general(x, wu, (((1,), (1,)), ((), ())),
                            preferred_element_type=jnp.float32)
        g = lax.dot_general(x, wg, (((1,), (1,)), ((), ())),
                            preferred_element_type=jnp.float32)
        a = (jax.nn.gelu(g) * h).astype(jnp.bfloat16)
        o_ref[...] = lax.dot_general(a, wd, (((1,), (0,)), ((), ())),
                                     preferred_element_type=jnp.float32)


def _grouped_ffn(x_perm, meta, wu, wg, wd):
    xmap = lambda b, mt: (jnp.minimum(b, mt[NB] - 1), 0)
    wmap = lambda b, mt: (mt[b], 0, 0)
    grid_spec = pltpu.PrefetchScalarGridSpec(
        num_scalar_prefetch=1,
        grid=(NB,),
        in_specs=[
            pl.BlockSpec((BLK, IN_DIM), xmap),
            pl.BlockSpec((1, HIDDEN_DIM, IN_DIM), wmap),
            pl.BlockSpec((1, HIDDEN_DIM, IN_DIM), wmap),
            pl.BlockSpec((1, HIDDEN_DIM, IN_DIM), wmap),
        ],
        out_specs=pl.BlockSpec((BLK, IN_DIM), xmap),
    )
    return pl.pallas_call(
        _ffn_body,
        grid_spec=grid_spec,
        out_shape=jax.ShapeDtypeStruct((P, IN_DIM), jnp.float32),
    )(meta, x_perm, wu, wg, wd)


def _combine_body(y0_ref, y1_ref, w_ref, o_ref):
    o_ref[...] = (y0_ref[...] * w_ref[:, 0:1] + y1_ref[...] * w_ref[:, 1:2])


def _combine(y_unperm, weights):
    rows = 256
    nblk = SEQ // rows
    return pl.pallas_call(
        _combine_body,
        grid=(nblk,),
        in_specs=[
            pl.BlockSpec((rows, IN_DIM), lambda i: (i, 0)),
            pl.BlockSpec((rows, IN_DIM), lambda i: (i + nblk, 0)),
            pl.BlockSpec((rows, TOP_K), lambda i: (i, 0)),
        ],
        out_specs=pl.BlockSpec((rows, IN_DIM), lambda i: (i, 0)),
        out_shape=jax.ShapeDtypeStruct((SEQ, IN_DIM), jnp.float32),
    )(y_unperm, y_unperm, weights)


_SR = SEQ // 128


def _routing_body(s0_ref, s1_ref, dt_ref, meta_ref):
    s0 = s0_ref[...]
    s1 = s1_ref[...]
    lane = lax.broadcasted_iota(jnp.int32, (_SR, 128), 1)
    row = lax.broadcasted_iota(jnp.int32, (_SR, 128), 0)
    ucol = lax.broadcasted_iota(jnp.int32, (128, 128), 1)
    urow = lax.broadcasted_iota(jnp.int32, (128, 128), 0)
    upper = (urow < ucol).astype(jnp.bfloat16)
    lrow = lax.broadcasted_iota(jnp.int32, (_SR, _SR), 0)
    lcol = lax.broadcasted_iota(jnp.int32, (_SR, _SR), 1)
    lower = (lrow > lcol).astype(jnp.bfloat16)
    ones = jnp.ones((128, 128), jnp.bfloat16)

    def eprefix(m):
        mb = m.astype(jnp.bfloat16)
        pl_lane = lax.dot_general(mb, upper, (((1,), (0,)), ((), ())),
                                  preferred_element_type=jnp.float32)
        tot_rep = lax.dot_general(mb, ones, (((1,), (0,)), ((), ())),
                                  preferred_element_type=jnp.float32)
        carry = lax.dot_general(lower, tot_rep.astype(jnp.bfloat16),
                                (((1,), (0,)), ((), ())),
                                preferred_element_type=jnp.float32)
        return pl_lane + carry

    d0 = jnp.zeros((_SR, 128), jnp.float32)
    d1 = jnp.zeros((_SR, 128), jnp.float32)
    pad_end_prev = jnp.int32(0)
    pad_ends = []
    for e in range(N_EXPERTS):
        m0 = (s0 == e)
        m1 = (s1 == e)
        p0 = eprefix(m0)
        p1 = eprefix(m1)
        cnt = (jnp.sum(m0.astype(jnp.float32))
               + jnp.sum(m1.astype(jnp.float32))).astype(jnp.int32)
        padded = ((cnt + BLK - 1) // BLK) * BLK
        pad_start = pad_end_prev
        pad_end_prev = pad_end_prev + padded
        pad_ends.append(pad_end_prev)
        base = pad_start.astype(jnp.float32) + p0 + p1
        d0 = jnp.where(m0, base, d0)
        d1 = jnp.where(m1, base + m0.astype(jnp.float32), d1)
    dt_ref[0:_SR, :] = d0.astype(jnp.int32)
    dt_ref[_SR:2 * _SR, :] = d1.astype(jnp.int32)
    total = pad_end_prev
    bv = lax.broadcasted_iota(jnp.int32, (1, 128), 1) * BLK
    be = jnp.zeros((1, 128), jnp.int32)
    e_last = jnp.int32(0)
    for e in range(N_EXPERTS):
        be = be + jnp.where(pad_ends[e] <= bv, 1, 0)
        e_last = e_last + jnp.where(pad_ends[e] <= total - 1, 1, 0)
    be = jnp.minimum(be, e_last)
    n_live = total // BLK
    lane1 = lax.broadcasted_iota(jnp.int32, (1, 128), 1)
    meta_ref[...] = jnp.where(lane1 == NB, n_live, be)


def _routing(selections):
    s0 = selections[:, 0].reshape(_SR, 128)
    s1 = selections[:, 1].reshape(_SR, 128)
    dt, meta = pl.pallas_call(
        _routing_body,
        out_shape=(jax.ShapeDtypeStruct((2 * _SR, 128), jnp.int32),
                   jax.ShapeDtypeStruct((1, 128), jnp.int32)),
    )(s0, s1)
    return dt, meta


def kernel(inp, weights, selections, up_proj, gate_proj, down_proj):
    dt, meta = _routing(selections)
    wu = up_proj.reshape(N_EXPERTS, HIDDEN_DIM, IN_DIM)
    wg = gate_proj.reshape(N_EXPERTS, HIDDEN_DIM, IN_DIM)
    wd = down_proj.reshape(N_EXPERTS, HIDDEN_DIM, IN_DIM)
    x_perm = _sc_scatter_tokens(inp, dt)
    y = _grouped_ffn(x_perm, meta.reshape(128), wu, wg, wd)
    y_unperm = _sc_gather_rows(y, dt, ROWS, IN_DIM)
    return _combine(y_unperm, weights)
